# baseline (speedup 1.0000x reference)
import functools

import jax
import jax.numpy as jnp
from jax import lax
from jax.experimental import pallas as pl
from jax.experimental.pallas import tpu as pltpu
from jax.experimental.pallas import tpu_sc as plsc

N = 10000
E = 320000
IN_DIM = 128
HIDDEN_DIM = 128
OUT_DIM = 1
NUM_RELS = 8

_NC = 2
_NS = 16
_NW = _NC * _NS
_CPT = 80
_GRP = 8
_CH = _NW * _CPT
_EPAD = _CH * 128
_ACC_ROWS = 10240

_MESH = plsc.VectorSubcoreMesh(
    core_axis_name="c", subcore_axis_name="s", num_cores=_NC, num_subcores=_NS)


def _proj_body(x_ref, w_ref, o_ref):
    o_ref[0] = jnp.dot(x_ref[...], w_ref[0], preferred_element_type=jnp.float32)


def _project(x, wstack):
    return pl.pallas_call(
        _proj_body,
        grid=(10, 9),
        in_specs=[
            pl.BlockSpec((N // 10, IN_DIM), lambda i, r: (i, 0)),
            pl.BlockSpec((1, IN_DIM, HIDDEN_DIM), lambda i, r: (r, 0, 0)),
        ],
        out_specs=pl.BlockSpec((1, N // 10, HIDDEN_DIM), lambda i, r: (r, i, 0)),
        out_shape=jax.ShapeDtypeStruct((9, N, HIDDEN_DIM), jnp.float32),
    )(x, wstack)


_RING = 2
_CPT0 = 160
_CPT1 = 0


@functools.partial(
    pl.kernel,
    out_type=jax.ShapeDtypeStruct((_NC, N, HIDDEN_DIM), jnp.float32),
    mesh=_MESH,
    compiler_params=pltpu.CompilerParams(needs_layout_passes=False),
    scratch_types=[
        pltpu.VMEM((2, _GRP, 128), jnp.int32),
        pltpu.VMEM((_GRP, 128), jnp.int32),
        pltpu.VMEM((2, _GRP, 128), jnp.int32),
        pltpu.VMEM((_RING, 128, HIDDEN_DIM), jnp.float32),
        pltpu.VMEM_SHARED((_ACC_ROWS, HIDDEN_DIM), jnp.float32),
        pltpu.SemaphoreType.DMA,
        pltpu.SemaphoreType.DMA,
        pltpu.SemaphoreType.DMA,
    ],
)
def _edge_agg1(table, srcb, etb, dstb, out, idx_v, et_v, dst_v, rows_v,
               acc_sh, sem_g, sem_s, sem_t):
    cid = lax.axis_index("c")
    sid = lax.axis_index("s")

    zero16 = jnp.zeros((16,), jnp.float32)

    @pl.loop(0, 128)
    def _zero_rows(r):
        for j in range(HIDDEN_DIM // 16):
            rows_v[0, r, pl.ds(j * 16, 16)] = zero16

    for t in range(5):
        pltpu.sync_copy(rows_v.at[0], acc_sh.at[pl.ds(sid * 640 + t * 128, 128)])

    cpt = lax.select(cid == 0, _CPT0, _CPT1)
    row0 = lax.select(cid == 0, sid * _CPT0, _NS * _CPT0 + sid * _CPT1)

    def _fire_stage(g):
        off = pl.multiple_of(row0 + g * _GRP, _GRP)
        slot = lax.rem(g, 2)
        pltpu.async_copy(srcb.at[pl.ds(off, _GRP)], idx_v.at[slot], sem_t)
        pltpu.async_copy(etb.at[pl.ds(off, _GRP)], et_v, sem_t)
        pltpu.async_copy(dstb.at[pl.ds(off, _GRP)], dst_v.at[slot], sem_t)

    def _wait_stage_and_flatten(g):
        slot = lax.rem(g, 2)
        for _ in range(3):
            pltpu.make_async_copy(srcb.at[pl.ds(0, _GRP)], et_v, sem_t).wait()
        for j in range(_GRP):
            for i in range(8):
                s16 = idx_v[slot, j, pl.ds(i * 16, 16)]
                e16 = et_v[j, pl.ds(i * 16, 16)]
                idx_v[slot, j, pl.ds(i * 16, 16)] = e16 * N + s16

    def _fire_gather(t):
        pltpu.async_copy(
            table.at[idx_v.at[lax.rem(lax.div(t, _GRP), 2)].at[lax.rem(t, _GRP)]],
            rows_v.at[lax.rem(t, _RING)], sem_g)

    def _wait_gather():
        pltpu.make_async_copy(table.at[idx_v.at[0].at[0]], rows_v.at[0],
                              sem_g).wait()

    def _fire_scatter(t):
        pltpu.async_copy(
            rows_v.at[lax.rem(t, _RING)],
            acc_sh.at[dst_v.at[lax.rem(lax.div(t, _GRP), 2)].at[lax.rem(t, _GRP)]],
            sem_s, add=True)

    def _wait_scatter():
        pltpu.make_async_copy(rows_v.at[0], acc_sh.at[dst_v.at[0].at[0]],
                              sem_s).wait()

    @pl.when(cpt > 0)
    def _prologue():
        _fire_stage(0)
        _wait_stage_and_flatten(0)
        _fire_gather(0)

    @pl.loop(0, cpt)
    def _chunk(t):
        @pl.when(t >= 1)
        def _():
            _wait_scatter()

        @pl.when(jnp.logical_and(lax.rem(t, _GRP) == 0, t + _GRP < cpt))
        def _():
            _fire_stage(lax.div(t, _GRP) + 1)

        @pl.when(jnp.logical_and(lax.rem(t, _GRP) == _GRP - 1, t + 1 < cpt))
        def _():
            _wait_stage_and_flatten(lax.div(t, _GRP) + 1)

        @pl.when(t < cpt - 1)
        def _():
            _fire_gather(t + 1)

        _wait_gather()
        _fire_scatter(t)

    @pl.when(cpt > 0)
    def _epilogue():
        _wait_scatter()

    plsc.subcore_barrier()

    @pl.when(sid < _NS - 1)
    def _full():
        pltpu.sync_copy(acc_sh.at[pl.ds(sid * 640, 640)],
                        out.at[cid, pl.ds(sid * 640, 640)])

    @pl.when(sid == _NS - 1)
    def _tail():
        pltpu.sync_copy(acc_sh.at[pl.ds(9600, 400)],
                        out.at[cid, pl.ds(9600, 400)])


def _h_body(a_ref, s_ref, b_ref, w_ref, o_ref):
    h = jnp.maximum(a_ref[0] + a_ref[1] + s_ref[0] + b_ref[...], 0.0)
    o_ref[...] = lax.dot_general(w_ref[...], h, (((1,), (1,)), ((), ())),
                                 preferred_element_type=jnp.float32)


def _layer2_proj(aggp, proj_all, bias1, wstack2):
    return pl.pallas_call(
        _h_body,
        grid=(1,),
        in_specs=[
            pl.BlockSpec((_NC, N, HIDDEN_DIM), lambda i: (0, 0, 0)),
            pl.BlockSpec((1, N, HIDDEN_DIM), lambda i: (8, 0, 0)),
            pl.BlockSpec((1, HIDDEN_DIM), lambda i: (0, 0)),
            pl.BlockSpec((16, HIDDEN_DIM), lambda i: (0, 0)),
        ],
        out_specs=pl.BlockSpec((16, N), lambda i: (0, 0)),
        out_shape=jax.ShapeDtypeStruct((16, N), jnp.float32),
    )(aggp, proj_all, bias1, wstack2)


@functools.partial(
    pl.kernel,
    out_type=jax.ShapeDtypeStruct((_NC, _ACC_ROWS), jnp.float32),
    mesh=_MESH,
    compiler_params=pltpu.CompilerParams(needs_layout_passes=False),
    scratch_types=[
        pltpu.VMEM((NUM_RELS * N,), jnp.float32),
        pltpu.VMEM((_GRP, 128), jnp.int32),
        pltpu.VMEM((_GRP, 128), jnp.int32),
        pltpu.VMEM((_GRP, 128), jnp.int32),
        pltpu.VMEM((_GRP, 128), jnp.float32),
        pltpu.VMEM((128,), jnp.float32),
        pltpu.VMEM_SHARED((_ACC_ROWS,), jnp.float32),
        pltpu.SemaphoreType.DMA,
    ],
)
def _edge_agg2(tab_hbm, srcb, etb, dstb, out, tab_v, src_v, et_v, dst_v, vals_v,
               z_v, acc_sh, sem):
    cid = lax.axis_index("c")
    sid = lax.axis_index("s")
    wid = sid * _NC + cid

    zero16 = jnp.zeros((16,), jnp.float32)
    for i in range(8):
        z_v[pl.ds(i * 16, 16)] = zero16
    for t in range(5):
        pltpu.sync_copy(z_v, acc_sh.at[pl.ds(sid * 640 + t * 128, 128)])
    pltpu.sync_copy(tab_hbm.at[pl.ds(0, NUM_RELS * N)], tab_v)
    plsc.subcore_barrier()

    @pl.loop(0, _CPT // _GRP)
    def _group(g):
        row0 = wid * _CPT + g * _GRP
        pltpu.sync_copy(srcb.at[pl.ds(row0, _GRP)], src_v)
        pltpu.sync_copy(etb.at[pl.ds(row0, _GRP)], et_v)
        pltpu.sync_copy(dstb.at[pl.ds(row0, _GRP)], dst_v)
        for j in range(_GRP):
            for i in range(8):
                s16 = src_v[j, pl.ds(i * 16, 16)]
                e16 = et_v[j, pl.ds(i * 16, 16)]
                vals_v[j, pl.ds(i * 16, 16)] = plsc.load_gather(
                    tab_v, [e16 * N + s16])
        copies = [
            pltpu.async_copy(vals_v.at[j], acc_sh.at[dst_v.at[j]], sem, add=True)
            for j in range(_GRP)
        ]
        for c in copies:
            c.wait()

    plsc.subcore_barrier()

    pltpu.sync_copy(acc_sh.at[pl.ds(sid * 640, 640)],
                    out.at[cid, pl.ds(sid * 640, 640)])


def _out_body(a_ref, p_ref, b_ref, o_ref):
    o_ref[...] = (a_ref[0:1, :N] + a_ref[1:2, :N] + p_ref[8:9, :]
                  + b_ref[...])


def _finalize(agg2p, proj2t, bias2):
    return pl.pallas_call(
        _out_body,
        grid=(1,),
        in_specs=[
            pl.BlockSpec((_NC, _ACC_ROWS), lambda i: (0, 0)),
            pl.BlockSpec((16, N), lambda i: (0, 0)),
            pl.BlockSpec((1, 1), lambda i: (0, 0)),
        ],
        out_specs=pl.BlockSpec((1, N), lambda i: (0, 0)),
        out_shape=jax.ShapeDtypeStruct((1, N), jnp.float32),
    )(agg2p, proj2t, bias2)


def kernel(x, edge_index, etypes, bases1, comp1, w_self1, bias1, bases2, comp2,
           w_self2, bias2):
    src = edge_index[0]
    dst = edge_index[1]

    W1 = jnp.einsum('rb,bio->rio', comp1, bases1)
    wstack1 = jnp.concatenate([W1, w_self1[None]], axis=0)
    W2 = jnp.einsum('rb,bio->rio', comp2, bases2)[..., 0]
    wstack2 = jnp.concatenate(
        [W2, w_self2.T, jnp.zeros((16 - NUM_RELS - 1, HIDDEN_DIM), jnp.float32)],
        axis=0)

    pad = _EPAD - E
    srcb = jnp.concatenate([src, jnp.zeros((pad,), jnp.int32)]).reshape(_CH, 128)
    etb = jnp.concatenate([etypes, jnp.zeros((pad,), jnp.int32)]).reshape(_CH, 128)
    dstb = jnp.concatenate([dst, jnp.full((pad,), N, jnp.int32)]).reshape(_CH, 128)

    proj_all = _project(x, wstack1)
    table1 = proj_all.reshape(9 * N, HIDDEN_DIM)
    aggp = _edge_agg1(table1, srcb, etb, dstb)
    proj2t = _layer2_proj(aggp, proj_all, bias1.reshape(1, HIDDEN_DIM), wstack2)
    agg2p = _edge_agg2(proj2t.reshape(16 * N), srcb, etb, dstb)
    out = _finalize(agg2p, proj2t, bias2.reshape(1, 1))
    return out.reshape(N, OUT_DIM)


# speedup vs baseline: 1.4895x; 1.4895x over previous
import functools

import jax
import jax.numpy as jnp
import numpy as np
from jax import lax
from jax.experimental import pallas as pl
from jax.experimental.pallas import tpu as pltpu
from jax.experimental.pallas import tpu_sc as plsc

N = 10000
E = 320000
IN_DIM = 128
HIDDEN_DIM = 128
OUT_DIM = 1
NUM_RELS = 8

_NC = 2
_NS = 16
_NW = _NC * _NS
_CPT = 80
_GRP = 8
_CH = _NW * _CPT
_EPAD = _CH * 128
_ACC_ROWS = 10240

_MESH = plsc.VectorSubcoreMesh(
    core_axis_name="c", subcore_axis_name="s", num_cores=_NC, num_subcores=_NS)

_PERM = np.empty((HIDDEN_DIM,), np.int32)
for _c in range(HIDDEN_DIM // 32):
    for _k in range(16):
        _PERM[32 * _c + 2 * _k] = 32 * _c + _k
        _PERM[32 * _c + 2 * _k + 1] = 32 * _c + 16 + _k


def _proj_body(x_ref, wp_ref, ws_ref, ob_ref, os_ref):
    xb = x_ref[...]
    for r in range(NUM_RELS):
        ob_ref[r] = jnp.dot(
            xb, wp_ref[r], preferred_element_type=jnp.float32
        ).astype(jnp.bfloat16)
    os_ref[...] = jnp.dot(xb, ws_ref[...], preferred_element_type=jnp.float32)


def _project(x, wperm, w_self):
    return pl.pallas_call(
        _proj_body,
        grid=(10,),
        in_specs=[
            pl.BlockSpec((N // 10, IN_DIM), lambda i: (i, 0)),
            pl.BlockSpec((NUM_RELS, IN_DIM, HIDDEN_DIM), lambda i: (0, 0, 0)),
            pl.BlockSpec((IN_DIM, HIDDEN_DIM), lambda i: (0, 0)),
        ],
        out_specs=[
            pl.BlockSpec((NUM_RELS, N // 10, HIDDEN_DIM), lambda i: (0, i, 0)),
            pl.BlockSpec((N // 10, HIDDEN_DIM), lambda i: (i, 0)),
        ],
        out_shape=[
            jax.ShapeDtypeStruct((NUM_RELS, N, HIDDEN_DIM), jnp.bfloat16),
            jax.ShapeDtypeStruct((N, HIDDEN_DIM), jnp.float32),
        ],
    )(x, wperm, w_self)


_RING = 3
_CPT0 = 80
_CPT1 = 80


@functools.partial(
    pl.kernel,
    out_type=jax.ShapeDtypeStruct((_NC, N, HIDDEN_DIM), jnp.float32),
    mesh=_MESH,
    compiler_params=pltpu.CompilerParams(needs_layout_passes=False,
                                         use_tc_tiling_on_sc=False),
    scratch_types=[
        pltpu.VMEM((2, _GRP, 128), jnp.int32),
        pltpu.VMEM((_GRP, 128), jnp.int32),
        pltpu.VMEM((2, _GRP, 128), jnp.int32),
        pltpu.VMEM((_RING, 128, HIDDEN_DIM), jnp.bfloat16),
        pltpu.VMEM((128, HIDDEN_DIM), jnp.float32),
        pltpu.VMEM_SHARED((_ACC_ROWS, HIDDEN_DIM), jnp.float32),
        pltpu.SemaphoreType.DMA,
        pltpu.SemaphoreType.DMA,
        pltpu.SemaphoreType.DMA,
    ],
)
def _edge_agg1(table, srcb, etb, dstb, out, idx_v, et_v, dst_v, rows_v, fbuf,
               acc_sh, sem_g, sem_s, sem_t):
    cid = lax.axis_index("c")
    sid = lax.axis_index("s")

    zero16 = jnp.zeros((16,), jnp.float32)

    @pl.loop(0, 128)
    def _zero_rows(r):
        for j in range(HIDDEN_DIM // 16):
            fbuf[r, pl.ds(j * 16, 16)] = zero16

    for t in range(5):
        pltpu.sync_copy(fbuf, acc_sh.at[pl.ds(sid * 640 + t * 128, 128)])

    cpt = lax.select(cid == 0, _CPT0, _CPT1)
    row0 = lax.select(cid == 0, sid * _CPT0, _NS * _CPT0 + sid * _CPT1)

    def _fire_stage(g):
        off = pl.multiple_of(row0 + g * _GRP, _GRP)
        slot = lax.rem(g, 2)
        pltpu.async_copy(srcb.at[pl.ds(off, _GRP)], idx_v.at[slot], sem_t)
        pltpu.async_copy(etb.at[pl.ds(off, _GRP)], et_v, sem_t)
        pltpu.async_copy(dstb.at[pl.ds(off, _GRP)], dst_v.at[slot], sem_t)

    def _wait_stage_and_flatten(g):
        slot = lax.rem(g, 2)
        for _ in range(3):
            pltpu.make_async_copy(srcb.at[pl.ds(0, _GRP)], et_v, sem_t).wait()
        for j in range(_GRP):
            for i in range(8):
                s16 = idx_v[slot, j, pl.ds(i * 16, 16)]
                e16 = et_v[j, pl.ds(i * 16, 16)]
                idx_v[slot, j, pl.ds(i * 16, 16)] = e16 * N + s16

    def _fire_gather(t):
        pltpu.async_copy(
            table.at[idx_v.at[lax.rem(lax.div(t, _GRP), 2)].at[lax.rem(t, _GRP)]],
            rows_v.at[lax.rem(t, _RING)], sem_g)

    def _wait_gather():
        pltpu.make_async_copy(table.at[idx_v.at[0].at[0]], rows_v.at[0],
                              sem_g).wait()

    def _convert(t):
        slot = lax.rem(t, _RING)

        @pl.loop(0, 128)
        def _row(r):
            for c in range(HIDDEN_DIM // 32):
                v = rows_v[slot, r, pl.ds(32 * c, 32)]
                a, b = plsc.unpack(v, format=plsc.PackFormat.INTERLEAVED)
                fbuf[r, pl.ds(32 * c, 16)] = a
                fbuf[r, pl.ds(32 * c + 16, 16)] = b

    def _fire_scatter(t):
        pltpu.async_copy(
            fbuf,
            acc_sh.at[dst_v.at[lax.rem(lax.div(t, _GRP), 2)].at[lax.rem(t, _GRP)]],
            sem_s, add=True)

    def _wait_scatter():
        pltpu.make_async_copy(fbuf, acc_sh.at[dst_v.at[0].at[0]],
                              sem_s).wait()

    @pl.when(cpt > 0)
    def _prologue():
        _fire_stage(0)
        _wait_stage_and_flatten(0)
        _fire_gather(0)
        _fire_gather(1)

    @pl.loop(0, cpt)
    def _chunk(t):
        @pl.when(t >= 1)
        def _():
            _wait_scatter()

        @pl.when(jnp.logical_and(lax.rem(t, _GRP) == 0, t + _GRP < cpt))
        def _():
            _fire_stage(lax.div(t, _GRP) + 1)

        @pl.when(jnp.logical_and(lax.rem(t, _GRP) == _GRP - 3, t + 3 < cpt))
        def _():
            _wait_stage_and_flatten(lax.div(t, _GRP) + 1)

        @pl.when(t + 2 < cpt)
        def _():
            _fire_gather(t + 2)

        _wait_gather()
        _convert(t)
        _fire_scatter(t)

    @pl.when(cpt > 0)
    def _epilogue():
        _wait_scatter()

    plsc.subcore_barrier()

    @pl.when(sid < _NS - 1)
    def _full():
        pltpu.sync_copy(acc_sh.at[pl.ds(sid * 640, 640)],
                        out.at[cid, pl.ds(sid * 640, 640)])

    @pl.when(sid == _NS - 1)
    def _tail():
        pltpu.sync_copy(acc_sh.at[pl.ds(9600, 400)],
                        out.at[cid, pl.ds(9600, 400)])


def _h_body(a_ref, s_ref, b_ref, w_ref, o_ref):
    h = jnp.maximum(a_ref[0] + a_ref[1] + s_ref[...] + b_ref[...], 0.0)
    o_ref[...] = lax.dot_general(w_ref[...], h, (((1,), (1,)), ((), ())),
                                 preferred_element_type=jnp.float32)


def _layer2_proj(aggp, self1, bias1, wstack2):
    return pl.pallas_call(
        _h_body,
        grid=(1,),
        in_specs=[
            pl.BlockSpec((_NC, N, HIDDEN_DIM), lambda i: (0, 0, 0)),
            pl.BlockSpec((N, HIDDEN_DIM), lambda i: (0, 0)),
            pl.BlockSpec((1, HIDDEN_DIM), lambda i: (0, 0)),
            pl.BlockSpec((16, HIDDEN_DIM), lambda i: (0, 0)),
        ],
        out_specs=pl.BlockSpec((16, N), lambda i: (0, 0)),
        out_shape=jax.ShapeDtypeStruct((16, N), jnp.float32),
    )(aggp, self1, bias1, wstack2)


@functools.partial(
    pl.kernel,
    out_type=jax.ShapeDtypeStruct((_NC, _ACC_ROWS), jnp.float32),
    mesh=_MESH,
    compiler_params=pltpu.CompilerParams(needs_layout_passes=False),
    scratch_types=[
        pltpu.VMEM((NUM_RELS * N,), jnp.float32),
        pltpu.VMEM((_GRP, 128), jnp.int32),
        pltpu.VMEM((_GRP, 128), jnp.int32),
        pltpu.VMEM((_GRP, 128), jnp.int32),
        pltpu.VMEM((_GRP, 128), jnp.float32),
        pltpu.VMEM((128,), jnp.float32),
        pltpu.VMEM_SHARED((_ACC_ROWS,), jnp.float32),
        pltpu.SemaphoreType.DMA,
    ],
)
def _edge_agg2(tab_hbm, srcb, etb, dstb, out, tab_v, src_v, et_v, dst_v, vals_v,
               z_v, acc_sh, sem):
    cid = lax.axis_index("c")
    sid = lax.axis_index("s")
    wid = sid * _NC + cid

    zero16 = jnp.zeros((16,), jnp.float32)
    for i in range(8):
        z_v[pl.ds(i * 16, 16)] = zero16
    for t in range(5):
        pltpu.sync_copy(z_v, acc_sh.at[pl.ds(sid * 640 + t * 128, 128)])
    pltpu.sync_copy(tab_hbm.at[pl.ds(0, NUM_RELS * N)], tab_v)
    plsc.subcore_barrier()

    @pl.loop(0, _CPT // _GRP)
    def _group(g):
        row0 = wid * _CPT + g * _GRP
        pltpu.sync_copy(srcb.at[pl.ds(row0, _GRP)], src_v)
        pltpu.sync_copy(etb.at[pl.ds(row0, _GRP)], et_v)
        pltpu.sync_copy(dstb.at[pl.ds(row0, _GRP)], dst_v)
        for j in range(_GRP):
            for i in range(8):
                s16 = src_v[j, pl.ds(i * 16, 16)]
                e16 = et_v[j, pl.ds(i * 16, 16)]
                vals_v[j, pl.ds(i * 16, 16)] = plsc.load_gather(
                    tab_v, [e16 * N + s16])
        copies = [
            pltpu.async_copy(vals_v.at[j], acc_sh.at[dst_v.at[j]], sem, add=True)
            for j in range(_GRP)
        ]
        for c in copies:
            c.wait()

    plsc.subcore_barrier()

    pltpu.sync_copy(acc_sh.at[pl.ds(sid * 640, 640)],
                    out.at[cid, pl.ds(sid * 640, 640)])


def _out_body(a_ref, p_ref, b_ref, o_ref):
    o_ref[...] = (a_ref[0:1, :N] + a_ref[1:2, :N] + p_ref[8:9, :]
                  + b_ref[...])


def _finalize(agg2p, proj2t, bias2):
    return pl.pallas_call(
        _out_body,
        grid=(1,),
        in_specs=[
            pl.BlockSpec((_NC, _ACC_ROWS), lambda i: (0, 0)),
            pl.BlockSpec((16, N), lambda i: (0, 0)),
            pl.BlockSpec((1, 1), lambda i: (0, 0)),
        ],
        out_specs=pl.BlockSpec((1, N), lambda i: (0, 0)),
        out_shape=jax.ShapeDtypeStruct((1, N), jnp.float32),
    )(agg2p, proj2t, bias2)


def kernel(x, edge_index, etypes, bases1, comp1, w_self1, bias1, bases2, comp2,
           w_self2, bias2):
    src = edge_index[0]
    dst = edge_index[1]

    W1 = jnp.einsum('rb,bio->rio', comp1, bases1)
    wperm = W1[:, :, _PERM]
    W2 = jnp.einsum('rb,bio->rio', comp2, bases2)[..., 0]
    wstack2 = jnp.concatenate(
        [W2, w_self2.T, jnp.zeros((16 - NUM_RELS - 1, HIDDEN_DIM), jnp.float32)],
        axis=0)

    pad = _EPAD - E
    srcb = jnp.concatenate([src, jnp.zeros((pad,), jnp.int32)]).reshape(_CH, 128)
    etb = jnp.concatenate([etypes, jnp.zeros((pad,), jnp.int32)]).reshape(_CH, 128)
    dstb = jnp.concatenate([dst, jnp.full((pad,), N, jnp.int32)]).reshape(_CH, 128)

    table_bf16, self1 = _project(x, wperm, w_self1)
    table1 = table_bf16.reshape(NUM_RELS * N, HIDDEN_DIM)
    aggp = _edge_agg1(table1, srcb, etb, dstb)
    proj2t = _layer2_proj(aggp, self1, bias1.reshape(1, HIDDEN_DIM), wstack2)
    agg2p = _edge_agg2(proj2t.reshape(16 * N), srcb, etb, dstb)
    out = _finalize(agg2p, proj2t, bias2.reshape(1, 1))
    return out.reshape(N, OUT_DIM)
